# baseline (speedup 1.0000x reference)
import functools

import jax
import jax.numpy as jnp
from jax import lax
from jax.experimental import pallas as pl
from jax.experimental.pallas import tpu as pltpu
from jax.experimental.pallas import tpu_sc as plsc

Z = 128
H = 128
N_NODES = 10000
N_EDGES = 320000

NC = 2
NS = 16
NW = NC * NS

K = 2
NE_K = N_EDGES // K
EPW = NE_K // NW
CHUNK = 40
NCHUNK = EPW // CHUNK

NPAD = 10240
NPT = NPAD // NS
ZROWS = 64
GBUF = 4
SBUF = 4

EBLK = 2000
NBLK = 1000

_mesh = plsc.VectorSubcoreMesh(
    core_axis_name="c", subcore_axis_name="s", num_cores=NC, num_subcores=NS
)


@functools.partial(
    pl.kernel,
    out_type=jax.ShapeDtypeStruct((NE_K, Z), jnp.float32),
    mesh=_mesh,
    scratch_types=[
        pltpu.VMEM((NCHUNK, CHUNK), jnp.int32),
        pltpu.VMEM((NCHUNK, CHUNK), jnp.int32),
        pltpu.VMEM((GBUF, CHUNK, Z), jnp.float32),
        pltpu.SemaphoreType.DMA((GBUF,)),
        pltpu.SemaphoreType.DMA((GBUF,)),
    ],
)
def _gather_add(ps_hbm, pr_hbm, s_hbm, r_hbm, out_hbm, idx_s, idx_r, rows,
                sem_g, sem_w):
    wid = lax.axis_index("s") * NC + lax.axis_index("c")
    base = wid * EPW
    pltpu.sync_copy(s_hbm.at[wid], idx_s)
    pltpu.sync_copy(r_hbm.at[wid], idx_r)

    def ps_copy(j, b):
        return pltpu.make_async_copy(ps_hbm.at[idx_s.at[j]], rows.at[b],
                                     sem_g.at[b])

    def pr_copy(j, b):
        return pltpu.make_async_copy(pr_hbm.at[idx_r.at[j]], rows.at[b],
                                     sem_g.at[b])

    def w_copy(j, b):
        return pltpu.make_async_copy(
            rows.at[b], out_hbm.at[pl.ds(base + j * CHUNK, CHUNK)], sem_w.at[b]
        )

    ps_copy(0, 0).start()

    def body(j, carry):
        b = lax.rem(j, GBUF)
        ps_copy(j, b).wait()
        nj = j + 1
        nb = lax.rem(nj, GBUF)

        @pl.when(nj < NCHUNK)
        def _():
            @pl.when(nj >= GBUF)
            def _():
                w_copy(nj - GBUF, nb).wait()

            ps_copy(nj, nb).start()

        pr_copy(j, b).start(add=True)
        pr_copy(j, b).wait()
        w_copy(j, b).start()
        return carry

    lax.fori_loop(0, NCHUNK, body, 0)
    for t in range(GBUF):
        k = NCHUNK - GBUF + t
        w_copy(k, k % GBUF).wait()


@functools.partial(
    pl.kernel,
    out_type=jax.ShapeDtypeStruct((NC, NPAD, Z), jnp.float32),
    mesh=_mesh,
    scratch_types=[
        pltpu.VMEM((NCHUNK, CHUNK), jnp.int32),
        pltpu.VMEM((SBUF, CHUNK, Z), jnp.float32),
        pltpu.VMEM((ZROWS, Z), jnp.float32),
        pltpu.VMEM_SHARED((NPAD, Z), jnp.float32),
        pltpu.SemaphoreType.DMA((SBUF,)),
        pltpu.SemaphoreType.DMA((SBUF,)),
    ],
)
def _scatter_add(upd_hbm, r_hbm, out_hbm, idx_r, rows, zbuf, acc, sem_l, sem_s):
    c = lax.axis_index("c")
    s = lax.axis_index("s")
    wid = s * NC + c
    base = wid * EPW

    zvec = jnp.zeros((16,), jnp.float32)

    def zrow(i, carry):
        for k in range(Z // 16):
            zbuf[i, pl.ds(k * 16, 16)] = zvec
        return carry

    lax.fori_loop(0, ZROWS, zrow, 0)
    for q in range(NPT // ZROWS):
        pltpu.sync_copy(zbuf, acc.at[pl.ds(s * NPT + q * ZROWS, ZROWS)])
    pltpu.sync_copy(r_hbm.at[wid], idx_r)
    plsc.subcore_barrier()

    def l_copy(j, b):
        return pltpu.make_async_copy(
            upd_hbm.at[pl.ds(base + j * CHUNK, CHUNK)], rows.at[b], sem_l.at[b]
        )

    def s_copy(j, b):
        return pltpu.make_async_copy(rows.at[b], acc.at[idx_r.at[j]],
                                     sem_s.at[b])

    l_copy(0, 0).start()

    def body(j, carry):
        b = lax.rem(j, SBUF)
        l_copy(j, b).wait()
        nj = j + 1
        nb = lax.rem(nj, SBUF)

        @pl.when(nj < NCHUNK)
        def _():
            @pl.when(nj >= SBUF)
            def _():
                s_copy(nj - SBUF, nb).wait()

            l_copy(nj, nb).start()

        s_copy(j, b).start(add=True)
        return carry

    lax.fori_loop(0, NCHUNK, body, 0)
    for t in range(SBUF):
        k = NCHUNK - SBUF + t
        s_copy(k, k % SBUF).wait()
    plsc.subcore_barrier()

    for q in range(NPT // ZROWS):
        off = s * NPT + q * ZROWS
        pltpu.sync_copy(acc.at[pl.ds(off, ZROWS)], zbuf)
        pltpu.sync_copy(zbuf, out_hbm.at[c].at[pl.ds(off, ZROWS)])


def _proj_body(nf_ref, w1a_ref, w1b_ref, b1_ref, ps_ref, pr_ref):
    nf = nf_ref[...]
    ps_ref[...] = (
        jnp.dot(nf, w1a_ref[...], preferred_element_type=jnp.float32) + b1_ref[...]
    )
    pr_ref[...] = jnp.dot(nf, w1b_ref[...], preferred_element_type=jnp.float32)


_proj = pl.pallas_call(
    _proj_body,
    grid=(N_NODES // NBLK,),
    in_specs=[
        pl.BlockSpec((NBLK, Z), lambda i: (i, 0)),
        pl.BlockSpec((Z, H), lambda i: (0, 0)),
        pl.BlockSpec((Z, H), lambda i: (0, 0)),
        pl.BlockSpec((1, H), lambda i: (0, 0)),
    ],
    out_specs=[
        pl.BlockSpec((NBLK, H), lambda i: (i, 0)),
        pl.BlockSpec((NBLK, H), lambda i: (i, 0)),
    ],
    out_shape=[
        jax.ShapeDtypeStruct((N_NODES, H), jnp.float32),
        jax.ShapeDtypeStruct((N_NODES, H), jnp.float32),
    ],
)


def _edge_body(g_ref, e_ref, w1c, w2, b2, w3, b3, upd_ref, new_ref):
    e = e_ref[...]
    h1 = jnp.maximum(
        g_ref[...] + jnp.dot(e, w1c[...], preferred_element_type=jnp.float32), 0.0
    )
    h2 = jnp.maximum(
        jnp.dot(h1, w2[...], preferred_element_type=jnp.float32) + b2[...], 0.0
    )
    upd = jnp.dot(h2, w3[...], preferred_element_type=jnp.float32) + b3[...]
    upd_ref[...] = upd
    new_ref[...] = upd + e


_edge_mlp = pl.pallas_call(
    _edge_body,
    grid=(NE_K // EBLK,),
    in_specs=[
        pl.BlockSpec((EBLK, H), lambda i: (i, 0)),
        pl.BlockSpec((EBLK, Z), lambda i: (i, 0)),
        pl.BlockSpec((Z, H), lambda i: (0, 0)),
        pl.BlockSpec((H, H), lambda i: (0, 0)),
        pl.BlockSpec((1, H), lambda i: (0, 0)),
        pl.BlockSpec((H, Z), lambda i: (0, 0)),
        pl.BlockSpec((1, Z), lambda i: (0, 0)),
    ],
    out_specs=[
        pl.BlockSpec((EBLK, Z), lambda i: (i, 0)),
        pl.BlockSpec((EBLK, Z), lambda i: (i, 0)),
    ],
    out_shape=[
        jax.ShapeDtypeStruct((NE_K, Z), jnp.float32),
        jax.ShapeDtypeStruct((NE_K, Z), jnp.float32),
    ],
)


def _node_body(nf_ref, p_refs, w1a, w1b, b1, w2, b2, w3, b3, out_ref):
    nf = nf_ref[...]
    agg = p_refs[0][...]
    for p in p_refs[1:]:
        agg = agg + p[...]
    h1 = jnp.maximum(
        jnp.dot(nf, w1a[...], preferred_element_type=jnp.float32)
        + jnp.dot(agg, w1b[...], preferred_element_type=jnp.float32)
        + b1[...],
        0.0,
    )
    h2 = jnp.maximum(
        jnp.dot(h1, w2[...], preferred_element_type=jnp.float32) + b2[...], 0.0
    )
    out_ref[...] = (
        jnp.dot(h2, w3[...], preferred_element_type=jnp.float32) + b3[...] + nf
    )


_node_mlp = pl.pallas_call(
    _node_body,
    grid=(N_NODES // NBLK,),
    in_specs=[
        pl.BlockSpec((NBLK, Z), lambda i: (i, 0)),
        [pl.BlockSpec((NBLK, Z), lambda i: (i, 0)) for _ in range(K * NC)],
        pl.BlockSpec((Z, H), lambda i: (0, 0)),
        pl.BlockSpec((Z, H), lambda i: (0, 0)),
        pl.BlockSpec((1, H), lambda i: (0, 0)),
        pl.BlockSpec((H, H), lambda i: (0, 0)),
        pl.BlockSpec((1, H), lambda i: (0, 0)),
        pl.BlockSpec((H, Z), lambda i: (0, 0)),
        pl.BlockSpec((1, Z), lambda i: (0, 0)),
    ],
    out_specs=pl.BlockSpec((NBLK, Z), lambda i: (i, 0)),
    out_shape=jax.ShapeDtypeStruct((N_NODES, Z), jnp.float32),
)


def kernel(node_features, edge_features, senders, receivers,
           eW1, eb1, eW2, eb2, eW3, eb3,
           nW1, nb1, nW2, nb2, nW3, nb3):
    s32 = senders.astype(jnp.int32).reshape(K, NW, NCHUNK, CHUNK)
    r32 = receivers.astype(jnp.int32).reshape(K, NW, NCHUNK, CHUNK)

    ps, pr = _proj(node_features, eW1[:Z], eW1[Z:2 * Z], eb1.reshape(1, H))

    w1c = eW1[2 * Z:]
    b2r = eb2.reshape(1, H)
    b3r = eb3.reshape(1, Z)

    upds, new_edges, parts = [], [], []
    for k in range(K):
        g = _gather_add(ps, pr, s32[k], r32[k])
        upd, new_e = _edge_mlp(
            g, edge_features[k * NE_K:(k + 1) * NE_K], w1c, eW2, b2r, eW3, b3r
        )
        upds.append(upd)
        new_edges.append(new_e)
        parts.append(_scatter_add(upd, r32[k]))

    p_list = [p[c, :N_NODES] for p in parts for c in range(NC)]
    new_node = _node_mlp(
        node_features, p_list,
        nW1[:Z], nW1[Z:], nb1.reshape(1, H), nW2, nb2.reshape(1, H), nW3,
        nb3.reshape(1, Z),
    )
    new_edge = jnp.concatenate(new_edges, axis=0) if K > 1 else new_edges[0]
    return new_node, new_edge


# speedup vs baseline: 1.3070x; 1.3070x over previous
import functools

import jax
import jax.numpy as jnp
from jax import lax
from jax.experimental import pallas as pl
from jax.experimental.pallas import tpu as pltpu
from jax.experimental.pallas import tpu_sc as plsc

Z = 128
H = 128
N_NODES = 10000
N_EDGES = 320000

NC = 2
NS = 16
NW = NC * NS
EPW = N_EDGES // NW
CHUNK = 80
NCHUNK = EPW // CHUNK
UNROLL = 5

NPAD = 10240
NPT = NPAD // NS
ZROWS = 64
GBUF = 4
SBUF = 2

EBLK = 2000
NBLK = 1000

_mesh = plsc.VectorSubcoreMesh(
    core_axis_name="c", subcore_axis_name="s", num_cores=NC, num_subcores=NS
)


@functools.partial(
    pl.kernel,
    out_type=jax.ShapeDtypeStruct((N_EDGES, Z), jnp.float32),
    mesh=_mesh,
    scratch_types=[
        pltpu.VMEM((NCHUNK, CHUNK), jnp.int32),
        pltpu.VMEM((NCHUNK, CHUNK), jnp.int32),
        pltpu.VMEM((GBUF, CHUNK, Z), jnp.float32),
        pltpu.SemaphoreType.DMA((GBUF,)),
        pltpu.SemaphoreType.DMA((GBUF,)),
    ],
)
def _gather_add(ps_hbm, pr_hbm, s_hbm, r_hbm, out_hbm, idx_s, idx_r, rows,
                sem_g, sem_w):
    wid = lax.axis_index("s") * NC + lax.axis_index("c")
    base = wid * EPW
    pltpu.sync_copy(s_hbm.at[wid], idx_s)
    pltpu.sync_copy(r_hbm.at[wid], idx_r)

    def ps_copy(j, b):
        return pltpu.make_async_copy(ps_hbm.at[idx_s.at[j]], rows.at[b],
                                     sem_g.at[b])

    def pr_copy(j, b):
        return pltpu.make_async_copy(pr_hbm.at[idx_r.at[j]], rows.at[b],
                                     sem_g.at[b])

    def w_copy(j, b):
        return pltpu.make_async_copy(
            rows.at[b], out_hbm.at[pl.ds(base + j * CHUNK, CHUNK)], sem_w.at[b]
        )

    ps_copy(0, 0).start()
    ps_copy(1, 1).start()

    def chunk_step(j):
        b = lax.rem(j, GBUF)
        ps_copy(j, b).wait()
        pr_copy(j, b).start(add=True)
        nj = j + 2
        nb = lax.rem(nj, GBUF)

        @pl.when(nj < NCHUNK)
        def _():
            @pl.when(nj >= GBUF)
            def _():
                w_copy(nj - GBUF, nb).wait()

            ps_copy(nj, nb).start()

        pr_copy(j, b).wait()
        w_copy(j, b).start()

    def body(jj, carry):
        for u in range(UNROLL):
            chunk_step(jj * UNROLL + u)
        return carry

    lax.fori_loop(0, NCHUNK // UNROLL, body, 0)
    for t in range(GBUF):
        k = NCHUNK - GBUF + t
        w_copy(k, k % GBUF).wait()


@functools.partial(
    pl.kernel,
    out_type=jax.ShapeDtypeStruct((NC, NPAD, Z), jnp.float32),
    mesh=_mesh,
    scratch_types=[
        pltpu.VMEM((NCHUNK, CHUNK), jnp.int32),
        pltpu.VMEM((SBUF, CHUNK, Z), jnp.float32),
        pltpu.VMEM((ZROWS, Z), jnp.float32),
        pltpu.VMEM_SHARED((NPAD, Z), jnp.float32),
        pltpu.SemaphoreType.DMA((SBUF,)),
        pltpu.SemaphoreType.DMA((SBUF,)),
    ],
)
def _scatter_add(upd_hbm, r_hbm, out_hbm, idx_r, rows, zbuf, acc, sem_l, sem_s):
    c = lax.axis_index("c")
    s = lax.axis_index("s")
    wid = s * NC + c
    base = wid * EPW

    zvec = jnp.zeros((16,), jnp.float32)

    def zrow(i, carry):
        for k in range(Z // 16):
            zbuf[i, pl.ds(k * 16, 16)] = zvec
        return carry

    lax.fori_loop(0, ZROWS, zrow, 0)
    for q in range(NPT // ZROWS):
        pltpu.sync_copy(zbuf, acc.at[pl.ds(s * NPT + q * ZROWS, ZROWS)])
    pltpu.sync_copy(r_hbm.at[wid], idx_r)
    plsc.subcore_barrier()

    def l_copy(j, b):
        return pltpu.make_async_copy(
            upd_hbm.at[pl.ds(base + j * CHUNK, CHUNK)], rows.at[b], sem_l.at[b]
        )

    def s_copy(j, b):
        return pltpu.make_async_copy(rows.at[b], acc.at[idx_r.at[j]],
                                     sem_s.at[b])

    l_copy(0, 0).start()

    def chunk_step(j):
        b = lax.rem(j, SBUF)
        l_copy(j, b).wait()
        nj = j + 1
        nb = lax.rem(nj, SBUF)

        @pl.when(nj < NCHUNK)
        def _():
            @pl.when(nj >= SBUF)
            def _():
                s_copy(nj - SBUF, nb).wait()

            l_copy(nj, nb).start()

        s_copy(j, b).start(add=True)

    def body(jj, carry):
        for u in range(UNROLL):
            chunk_step(jj * UNROLL + u)
        return carry

    lax.fori_loop(0, NCHUNK // UNROLL, body, 0)
    for t in range(SBUF):
        k = NCHUNK - SBUF + t
        s_copy(k, k % SBUF).wait()
    plsc.subcore_barrier()

    for q in range(NPT // ZROWS):
        off = s * NPT + q * ZROWS
        pltpu.sync_copy(acc.at[pl.ds(off, ZROWS)], zbuf)
        pltpu.sync_copy(zbuf, out_hbm.at[c].at[pl.ds(off, ZROWS)])


def _proj_body(nf_ref, w1a_ref, w1b_ref, b1_ref, ps_ref, pr_ref):
    nf = nf_ref[...]
    ps_ref[...] = (
        jnp.dot(nf, w1a_ref[...], preferred_element_type=jnp.float32) + b1_ref[...]
    )
    pr_ref[...] = jnp.dot(nf, w1b_ref[...], preferred_element_type=jnp.float32)


_proj = pl.pallas_call(
    _proj_body,
    grid=(N_NODES // NBLK,),
    in_specs=[
        pl.BlockSpec((NBLK, Z), lambda i: (i, 0)),
        pl.BlockSpec((Z, H), lambda i: (0, 0)),
        pl.BlockSpec((Z, H), lambda i: (0, 0)),
        pl.BlockSpec((1, H), lambda i: (0, 0)),
    ],
    out_specs=[
        pl.BlockSpec((NBLK, H), lambda i: (i, 0)),
        pl.BlockSpec((NBLK, H), lambda i: (i, 0)),
    ],
    out_shape=[
        jax.ShapeDtypeStruct((N_NODES, H), jnp.float32),
        jax.ShapeDtypeStruct((N_NODES, H), jnp.float32),
    ],
)


def _edge_body(g_ref, e_ref, w1c, w2, b2, w3, b3, upd_ref, new_ref):
    e = e_ref[...]
    h1 = jnp.maximum(
        g_ref[...] + jnp.dot(e, w1c[...], preferred_element_type=jnp.float32), 0.0
    )
    h2 = jnp.maximum(
        jnp.dot(h1, w2[...], preferred_element_type=jnp.float32) + b2[...], 0.0
    )
    upd = jnp.dot(h2, w3[...], preferred_element_type=jnp.float32) + b3[...]
    upd_ref[...] = upd
    new_ref[...] = upd + e


_edge_mlp = pl.pallas_call(
    _edge_body,
    grid=(N_EDGES // EBLK,),
    in_specs=[
        pl.BlockSpec((EBLK, H), lambda i: (i, 0)),
        pl.BlockSpec((EBLK, Z), lambda i: (i, 0)),
        pl.BlockSpec((Z, H), lambda i: (0, 0)),
        pl.BlockSpec((H, H), lambda i: (0, 0)),
        pl.BlockSpec((1, H), lambda i: (0, 0)),
        pl.BlockSpec((H, Z), lambda i: (0, 0)),
        pl.BlockSpec((1, Z), lambda i: (0, 0)),
    ],
    out_specs=[
        pl.BlockSpec((EBLK, Z), lambda i: (i, 0)),
        pl.BlockSpec((EBLK, Z), lambda i: (i, 0)),
    ],
    out_shape=[
        jax.ShapeDtypeStruct((N_EDGES, Z), jnp.float32),
        jax.ShapeDtypeStruct((N_EDGES, Z), jnp.float32),
    ],
)


def _node_body(nf_ref, p0, p1, w1a, w1b, b1, w2, b2, w3, b3, out_ref):
    nf = nf_ref[...]
    agg = p0[...] + p1[...]
    h1 = jnp.maximum(
        jnp.dot(nf, w1a[...], preferred_element_type=jnp.float32)
        + jnp.dot(agg, w1b[...], preferred_element_type=jnp.float32)
        + b1[...],
        0.0,
    )
    h2 = jnp.maximum(
        jnp.dot(h1, w2[...], preferred_element_type=jnp.float32) + b2[...], 0.0
    )
    out_ref[...] = (
        jnp.dot(h2, w3[...], preferred_element_type=jnp.float32) + b3[...] + nf
    )


_node_mlp = pl.pallas_call(
    _node_body,
    grid=(N_NODES // NBLK,),
    in_specs=[
        pl.BlockSpec((NBLK, Z), lambda i: (i, 0)),
        pl.BlockSpec((NBLK, Z), lambda i: (i, 0)),
        pl.BlockSpec((NBLK, Z), lambda i: (i, 0)),
        pl.BlockSpec((Z, H), lambda i: (0, 0)),
        pl.BlockSpec((Z, H), lambda i: (0, 0)),
        pl.BlockSpec((1, H), lambda i: (0, 0)),
        pl.BlockSpec((H, H), lambda i: (0, 0)),
        pl.BlockSpec((1, H), lambda i: (0, 0)),
        pl.BlockSpec((H, Z), lambda i: (0, 0)),
        pl.BlockSpec((1, Z), lambda i: (0, 0)),
    ],
    out_specs=pl.BlockSpec((NBLK, Z), lambda i: (i, 0)),
    out_shape=jax.ShapeDtypeStruct((N_NODES, Z), jnp.float32),
)


def kernel(node_features, edge_features, senders, receivers,
           eW1, eb1, eW2, eb2, eW3, eb3,
           nW1, nb1, nW2, nb2, nW3, nb3):
    s32 = senders.astype(jnp.int32).reshape(NW, NCHUNK, CHUNK)
    r32 = receivers.astype(jnp.int32).reshape(NW, NCHUNK, CHUNK)

    ps, pr = _proj(node_features, eW1[:Z], eW1[Z:2 * Z], eb1.reshape(1, H))
    g = _gather_add(ps, pr, s32, r32)
    upd, new_edge = _edge_mlp(
        g, edge_features, eW1[2 * Z:], eW2, eb2.reshape(1, H), eW3,
        eb3.reshape(1, Z),
    )
    parts = _scatter_add(upd, r32)
    new_node = _node_mlp(
        node_features, parts[0, :N_NODES], parts[1, :N_NODES],
        nW1[:Z], nW1[Z:], nb1.reshape(1, H), nW2, nb2.reshape(1, H), nW3,
        nb3.reshape(1, Z),
    )
    return new_node, new_edge
